# baseline (speedup 1.0000x reference)
import functools

import jax
import jax.numpy as jnp
from jax import lax
from jax.experimental import pallas as pl
from jax.experimental.pallas import tpu as pltpu
from jax.experimental.pallas import tpu_sc as plsc

N = 320000
D = 128
NB = 10000

NC = 2
NS = 16
NW = NC * NS
C = N // NW
NBP = 10240
BPW = NBP // NW
L = 16

NEG = float("-inf")

NSTREAM = 4
BLK = 12800
SUB = BLK // NSTREAM


def _matvec_body(*refs):
    z_refs = refs[:NSTREAM]
    w_ref, b_ref, out_ref = refs[NSTREAM:]
    w = w_ref[...]
    subs = []
    for z_ref in z_refs:
        x = z_ref[...]
        s = jax.lax.dot_general(
            x, w, (((1,), (0,)), ((), ())),
            preferred_element_type=jnp.float32)
        subs.append(s)
    out_ref[...] = jnp.concatenate(subs, axis=0) + b_ref[0, 0]


def _scores(z, W, b):
    wcol = W.reshape(D, 1)
    b2 = b.reshape(1, 1)
    zspecs = [
        pl.BlockSpec((SUB, D), functools.partial(
            lambda j, i: (NSTREAM * i + j, 0), j))
        for j in range(NSTREAM)
    ]
    out = pl.pallas_call(
        _matvec_body,
        grid=(N // BLK,),
        in_specs=zspecs + [
            pl.BlockSpec((D, 1), lambda i: (0, 0)),
            pl.BlockSpec((1, 1), lambda i: (0, 0)),
        ],
        out_specs=pl.BlockSpec((BLK, 1), lambda i: (i, 0)),
        out_shape=jax.ShapeDtypeStruct((N, 1), jnp.float32),
    )(*([z] * NSTREAM), wcol, b2)
    return out.reshape(N)


MV_NBUF = 8
MV_CHUNK = 4000
MV_K = N // MV_CHUNK


def _mv_manual_body(z_hbm, w_ref, b_ref, out_hbm, z_buf, o_buf, in_sems, out_sems):
    w = w_ref[...]
    bb = b_ref[0, 0]

    def start_in(k, slot):
        pltpu.make_async_copy(
            z_hbm.at[pl.ds(k * MV_CHUNK, MV_CHUNK), :],
            z_buf.at[slot], in_sems.at[slot]).start()

    for s in range(MV_NBUF):
        start_in(s, s)

    def outer(o, carry):
        for bslot in range(MV_NBUF):
            k = o * MV_NBUF + bslot
            pltpu.make_async_copy(
                z_hbm.at[pl.ds(k * MV_CHUNK, MV_CHUNK), :],
                z_buf.at[bslot], in_sems.at[bslot]).wait()

            @pl.when(o > 0)
            def _():
                pltpu.make_async_copy(
                    o_buf.at[bslot],
                    out_hbm.at[pl.ds(k * MV_CHUNK, MV_CHUNK), :],
                    out_sems.at[bslot]).wait()

            x = z_buf[bslot]
            s = jax.lax.dot_general(
                x, w, (((1,), (0,)), ((), ())),
                preferred_element_type=jnp.float32)
            o_buf[bslot] = s + bb
            pltpu.make_async_copy(
                o_buf.at[bslot],
                out_hbm.at[pl.ds(k * MV_CHUNK, MV_CHUNK), :],
                out_sems.at[bslot]).start()

            @pl.when(k + MV_NBUF < MV_K)
            def _():
                start_in(k + MV_NBUF, bslot)
        return carry

    lax.fori_loop(0, MV_K // MV_NBUF, outer, 0)
    for bslot in range(MV_NBUF):
        k = (MV_K // MV_NBUF - 1) * MV_NBUF + bslot
        pltpu.make_async_copy(
            o_buf.at[bslot],
            out_hbm.at[pl.ds(k * MV_CHUNK, MV_CHUNK), :],
            out_sems.at[bslot]).wait()


def _scores_manual(z, W, b):
    wcol = W.reshape(D, 1)
    b2 = b.reshape(1, 1)
    out = pl.pallas_call(
        _mv_manual_body,
        in_specs=[
            pl.BlockSpec(memory_space=pltpu.HBM),
            pl.BlockSpec(memory_space=pltpu.VMEM),
            pl.BlockSpec(memory_space=pltpu.VMEM),
        ],
        out_specs=pl.BlockSpec(memory_space=pltpu.HBM),
        out_shape=jax.ShapeDtypeStruct((N, 1), jnp.float32),
        scratch_shapes=[
            pltpu.VMEM((MV_NBUF, MV_CHUNK, D), jnp.float32),
            pltpu.VMEM((MV_NBUF, MV_CHUNK, 1), jnp.float32),
            pltpu.SemaphoreType.DMA((MV_NBUF,)),
            pltpu.SemaphoreType.DMA((MV_NBUF,)),
        ],
    )(z, wcol, b2)
    return out.reshape(N)


_MESH = plsc.VectorSubcoreMesh(core_axis_name="c", subcore_axis_name="s")
_SC_PARAMS = pltpu.CompilerParams(
    needs_layout_passes=False, use_tc_tiling_on_sc=False)


def _take(v, idx):
    return jnp.take_along_axis(v, idx, axis=0, mode="promise_in_bounds")


PRB_CH = 200
PRB_N = 24


@functools.partial(
    pl.kernel,
    mesh=_MESH,
    compiler_params=_SC_PARAMS,
    out_type=jax.ShapeDtypeStruct((NW, NBP), jnp.float32),
    scratch_types=[
        pltpu.VMEM((C,), jnp.float32),
        pltpu.VMEM((C,), jnp.int32),
        pltpu.VMEM((NBP,), jnp.float32),
        pltpu.VMEM((2, PRB_CH, D), jnp.float32),
        pltpu.SemaphoreType.DMA((2,)),
    ],
)
def _segmax_part(scores_hbm, seg_hbm, z_hbm, out_hbm, sc_v, seg_v, m_v, zp_v, zp_sems):
    wid = lax.axis_index("s") * NC + lax.axis_index("c")
    base = pl.multiple_of(wid * C, 8)

    def zdma(k, slot):
        return pltpu.make_async_copy(
            z_hbm.at[pl.ds(base + k * PRB_CH, PRB_CH), :],
            zp_v.at[slot], zp_sems.at[slot])

    zdma(0, 0).start()
    zdma(1, 1).start()
    for k in range(PRB_N):
        zdma(k, k % 2).wait()
        if k + 2 < PRB_N:
            zdma(k + 2, k % 2).start()

    pltpu.sync_copy(scores_hbm.at[pl.ds(base, C)], sc_v)
    pltpu.sync_copy(seg_hbm.at[pl.ds(base, C)], seg_v)

    neg = jnp.full((L,), NEG, jnp.float32)

    def init_body(i, carry):
        m_v[pl.ds(pl.multiple_of(i * L, L), L)] = neg
        return carry

    lax.fori_loop(0, NBP // L, init_body, 0, unroll=8)

    lane = lax.iota(jnp.int32, L)
    last_lane = lane == (L - 1)
    up1 = jnp.minimum(lane + 1, L - 1)

    def body(i, carry):
        off = pl.multiple_of(i * L, L)
        g = seg_v[pl.ds(off, L)]
        v = sc_v[pl.ds(off, L)]
        for s in (1, 2, 4, 8):
            idx = jnp.maximum(lane - s, 0)
            vs = _take(v, idx)
            gs = _take(g, idx)
            v = jnp.where((gs == g) & (lane >= s), jnp.maximum(v, vs), v)
        g_next = _take(g, up1)
        is_last = (g_next != g) | last_lane
        cur = plsc.load_gather(m_v, [g], mask=is_last)
        plsc.store_scatter(m_v, [g], jnp.maximum(cur, v), mask=is_last)
        return carry

    lax.fori_loop(0, C // L, body, 0)
    pltpu.sync_copy(m_v, out_hbm.at[wid])


@functools.partial(
    pl.kernel,
    mesh=_MESH,
    compiler_params=_SC_PARAMS,
    out_type=jax.ShapeDtypeStruct((NBP,), jnp.float32),
    scratch_types=[
        pltpu.VMEM((NW, BPW), jnp.float32),
        pltpu.VMEM((BPW,), jnp.float32),
    ],
)
def _segmax_merge(parts_hbm, out_hbm, blk_v, acc_v):
    wid = lax.axis_index("s") * NC + lax.axis_index("c")
    lo = pl.multiple_of(wid * BPW, 8)
    pltpu.sync_copy(parts_hbm.at[:, pl.ds(lo, BPW)], blk_v)

    def body(j, carry):
        off = pl.multiple_of(j * L, L)
        acc = jnp.full((L,), NEG, jnp.float32)
        for r in range(NW):
            acc = jnp.maximum(acc, blk_v[r, pl.ds(off, L)])
        acc_v[pl.ds(off, L)] = acc
        return carry

    lax.fori_loop(0, BPW // L, body, 0)
    pltpu.sync_copy(acc_v, out_hbm.at[pl.ds(lo, BPW)])


def kernel(z_ins, bag_idx, W, b):
    seg = bag_idx.astype(jnp.int32)
    scores = _scores_manual(z_ins, W, b)
    parts = _segmax_part(scores, seg, z_ins)
    merged = _segmax_merge(parts)
    M = merged[:NB][:, None]
    return (M, None, scores)


# speedup vs baseline: 1.9235x; 1.9235x over previous
import functools

import jax
import jax.numpy as jnp
from jax import lax
from jax.experimental import pallas as pl
from jax.experimental.pallas import tpu as pltpu
from jax.experimental.pallas import tpu_sc as plsc

N = 320000
D = 128
NB = 10000

NC = 2
NS = 16
NW = NC * NS
C = N // NW
NBP = 10240
BPW = NBP // NW
L = 16

RCH = 400
NCHK = C // RCH
GPC = RCH // L

NEG = float("-inf")

_MESH = plsc.VectorSubcoreMesh(core_axis_name="c", subcore_axis_name="s")
_SC_PARAMS = pltpu.CompilerParams(
    needs_layout_passes=False, use_tc_tiling_on_sc=False)


def _take(v, idx):
    return jnp.take_along_axis(v, idx, axis=0, mode="promise_in_bounds")


@functools.partial(
    pl.kernel,
    mesh=_MESH,
    compiler_params=_SC_PARAMS,
    out_type=(
        jax.ShapeDtypeStruct((N,), jnp.float32),
        jax.ShapeDtypeStruct((NW, NBP), jnp.float32),
    ),
    scratch_types=[
        pltpu.VMEM((2 * RCH, D), jnp.float32),
        pltpu.VMEM((2, RCH), jnp.int32),
        pltpu.VMEM((2, RCH), jnp.float32),
        pltpu.VMEM((NBP,), jnp.float32),
        pltpu.VMEM((D,), jnp.float32),
        pltpu.VMEM((L,), jnp.float32),
        pltpu.SemaphoreType.DMA((2,)),
        pltpu.SemaphoreType.DMA((2,)),
        pltpu.SemaphoreType.DMA((2,)),
    ],
)
def _sc_fused(z_hbm, seg_hbm, w_hbm, b_hbm, out_s_hbm, out_m_hbm,
              zb, segb, sb, m_v, w_v, b_v, zsem, gsem, osem):
    wid = lax.axis_index("s") * NC + lax.axis_index("c")
    base = pl.multiple_of(wid * C, 8)

    pltpu.sync_copy(w_hbm, w_v)
    pltpu.sync_copy(b_hbm, b_v)

    def z_dma(chunk, slot):
        return pltpu.make_async_copy(
            z_hbm.at[pl.ds(base + chunk * RCH, RCH), :],
            zb.at[pl.ds(slot * RCH, RCH), :], zsem.at[slot])

    def seg_dma(chunk, slot):
        return pltpu.make_async_copy(
            seg_hbm.at[pl.ds(base + chunk * RCH, RCH)],
            segb.at[slot], gsem.at[slot])

    def out_dma(chunk, slot):
        return pltpu.make_async_copy(
            sb.at[slot],
            out_s_hbm.at[pl.ds(base + chunk * RCH, RCH)], osem.at[slot])

    z_dma(0, 0).start()
    seg_dma(0, 0).start()
    z_dma(1, 1).start()
    seg_dma(1, 1).start()

    neg = jnp.full((L,), NEG, jnp.float32)

    def init_body(i, carry):
        m_v[pl.ds(pl.multiple_of(i * L, L), L)] = neg
        return carry

    lax.fori_loop(0, NBP // L, init_body, 0, unroll=8)

    wv = [w_v[pl.ds(16 * j, L)] for j in range(D // L)]
    bvec = b_v[...]
    lane = lax.iota(jnp.int32, L)
    last_lane = lane == (L - 1)
    up1 = jnp.minimum(lane + 1, L - 1)

    def body(g, carry):
        chunk = g // GPC
        gin = g - chunk * GPC
        slot = lax.rem(chunk, 2)

        for s in (0, 1):
            @pl.when((gin == 0) & (slot == s))
            def _():
                z_dma(chunk, s).wait()
                seg_dma(chunk, s).wait()

            @pl.when((gin == 0) & (slot == s) & (chunk >= 2))
            def _():
                out_dma(chunk - 2, s).wait()

        rowbase = slot * RCH + gin * L
        v = neg
        for l in range(L):
            row = rowbase + l
            t = zb[row, pl.ds(0, L)] * wv[0]
            for j in range(1, D // L):
                t = t + zb[row, pl.ds(16 * j, L)] * wv[j]
            v = jnp.where(lane == l, jnp.sum(t), v)
        v = v + bvec
        sb[slot, pl.ds(pl.multiple_of(gin * L, L), L)] = v

        gid = segb[slot, pl.ds(pl.multiple_of(gin * L, L), L)]
        for s in (1, 2, 4, 8):
            idx = jnp.maximum(lane - s, 0)
            vs = _take(v, idx)
            gs = _take(gid, idx)
            v = jnp.where((gs == gid) & (lane >= s), jnp.maximum(v, vs), v)
        g_next = _take(gid, up1)
        is_last = (g_next != gid) | last_lane
        cur = plsc.load_gather(m_v, [gid], mask=is_last)
        plsc.store_scatter(m_v, [gid], jnp.maximum(cur, v), mask=is_last)

        for s in (0, 1):
            @pl.when((gin == GPC - 1) & (slot == s))
            def _():
                out_dma(chunk, s).start()

            @pl.when((gin == GPC - 1) & (slot == s) & (chunk + 2 < NCHK))
            def _():
                z_dma(chunk + 2, s).start()
                seg_dma(chunk + 2, s).start()
        return carry

    lax.fori_loop(0, NCHK * GPC, body, 0)

    out_dma(NCHK - 2, (NCHK - 2) % 2).wait()
    out_dma(NCHK - 1, (NCHK - 1) % 2).wait()
    pltpu.sync_copy(m_v, out_m_hbm.at[wid])


@functools.partial(
    pl.kernel,
    mesh=_MESH,
    compiler_params=_SC_PARAMS,
    out_type=jax.ShapeDtypeStruct((NBP,), jnp.float32),
    scratch_types=[
        pltpu.VMEM((NW, BPW), jnp.float32),
        pltpu.VMEM((BPW,), jnp.float32),
    ],
)
def _segmax_merge(parts_hbm, out_hbm, blk_v, acc_v):
    wid = lax.axis_index("s") * NC + lax.axis_index("c")
    lo = pl.multiple_of(wid * BPW, 8)
    pltpu.sync_copy(parts_hbm.at[:, pl.ds(lo, BPW)], blk_v)

    def body(j, carry):
        off = pl.multiple_of(j * L, L)
        acc = jnp.full((L,), NEG, jnp.float32)
        for r in range(NW):
            acc = jnp.maximum(acc, blk_v[r, pl.ds(off, L)])
        acc_v[pl.ds(off, L)] = acc
        return carry

    lax.fori_loop(0, BPW // L, body, 0)
    pltpu.sync_copy(acc_v, out_hbm.at[pl.ds(lo, BPW)])


def kernel(z_ins, bag_idx, W, b):
    seg = bag_idx.astype(jnp.int32)
    w1 = W.reshape(D)
    b16 = jnp.broadcast_to(b, (L,))
    scores, parts = _sc_fused(z_ins, seg, w1, b16)
    merged = _segmax_merge(parts)
    M = merged[:NB][:, None]
    return (M, None, scores)


# speedup vs baseline: 2.0810x; 1.0819x over previous
import functools

import jax
import jax.numpy as jnp
from jax import lax
from jax.experimental import pallas as pl
from jax.experimental.pallas import tpu as pltpu
from jax.experimental.pallas import tpu_sc as plsc

N = 320000
D = 128
NB = 10000

NC = 2
NS = 16
NW = NC * NS
C = N // NW
NBP = 10240
BPW = NBP // NW
L = 16

RCH = 400
NCHK = C // RCH
GPC = RCH // L

NEG = float("-inf")

_MESH = plsc.VectorSubcoreMesh(core_axis_name="c", subcore_axis_name="s")
_SC_PARAMS = pltpu.CompilerParams(
    needs_layout_passes=False, use_tc_tiling_on_sc=False)


def _take(v, idx):
    return jnp.take_along_axis(v, idx, axis=0, mode="promise_in_bounds")


@functools.partial(
    pl.kernel,
    mesh=_MESH,
    compiler_params=_SC_PARAMS,
    out_type=(
        jax.ShapeDtypeStruct((N,), jnp.float32),
        jax.ShapeDtypeStruct((NW, NBP), jnp.float32),
    ),
    scratch_types=[
        pltpu.VMEM((2 * RCH, D), jnp.float32),
        pltpu.VMEM((2, RCH), jnp.int32),
        pltpu.VMEM((2, RCH), jnp.float32),
        pltpu.VMEM((NBP,), jnp.float32),
        pltpu.VMEM((D,), jnp.float32),
        pltpu.VMEM((L,), jnp.float32),
        pltpu.SemaphoreType.DMA((2,)),
        pltpu.SemaphoreType.DMA((2,)),
        pltpu.SemaphoreType.DMA((2,)),
    ],
)
def _sc_fused(z_hbm, seg_hbm, w_hbm, b_hbm, out_s_hbm, out_m_hbm,
              zb, segb, sb, m_v, w_v, b_v, zsem, gsem, osem):
    wid = lax.axis_index("s") * NC + lax.axis_index("c")
    base = pl.multiple_of(wid * C, 8)

    pltpu.sync_copy(w_hbm, w_v)
    pltpu.sync_copy(b_hbm, b_v)

    def z_dma(chunk, slot):
        return pltpu.make_async_copy(
            z_hbm.at[pl.ds(base + chunk * RCH, RCH), :],
            zb.at[pl.ds(slot * RCH, RCH), :], zsem.at[slot])

    def seg_dma(chunk, slot):
        return pltpu.make_async_copy(
            seg_hbm.at[pl.ds(base + chunk * RCH, RCH)],
            segb.at[slot], gsem.at[slot])

    def out_dma(chunk, slot):
        return pltpu.make_async_copy(
            sb.at[slot],
            out_s_hbm.at[pl.ds(base + chunk * RCH, RCH)], osem.at[slot])

    z_dma(0, 0).start()
    seg_dma(0, 0).start()
    z_dma(1, 1).start()
    seg_dma(1, 1).start()

    neg = jnp.full((L,), NEG, jnp.float32)

    def init_body(i, carry):
        m_v[pl.ds(pl.multiple_of(i * L, L), L)] = neg
        return carry

    lax.fori_loop(0, NBP // L, init_body, 0, unroll=8)

    wv = [w_v[pl.ds(16 * j, L)] for j in range(D // L)]
    bvec = b_v[...]
    lane = lax.iota(jnp.int32, L)
    last_lane = lane == (L - 1)
    up1 = jnp.minimum(lane + 1, L - 1)

    def chunk_body(chunk, carry):
        slot = lax.rem(chunk, 2)

        for s in (0, 1):
            @pl.when(slot == s)
            def _():
                z_dma(chunk, s).wait()
                seg_dma(chunk, s).wait()

            @pl.when((slot == s) & (chunk >= 2))
            def _():
                out_dma(chunk - 2, s).wait()

        def group_body(gin, carry2):
            rowbase = slot * RCH + gin * L
            v = neg
            for l in range(L):
                row = rowbase + l
                t = zb[row, pl.ds(0, L)] * wv[0]
                for j in range(1, D // L):
                    t = t + zb[row, pl.ds(16 * j, L)] * wv[j]
                v = jnp.where(lane == l, jnp.sum(t), v)
            v = v + bvec
            sb[slot, pl.ds(pl.multiple_of(gin * L, L), L)] = v

            gid = segb[slot, pl.ds(pl.multiple_of(gin * L, L), L)]
            for s in (1, 2, 4, 8):
                idx = jnp.maximum(lane - s, 0)
                vs = _take(v, idx)
                gs = _take(gid, idx)
                v = jnp.where((gs == gid) & (lane >= s),
                              jnp.maximum(v, vs), v)
            g_next = _take(gid, up1)
            is_last = (g_next != gid) | last_lane
            cur = plsc.load_gather(m_v, [gid], mask=is_last)
            plsc.store_scatter(m_v, [gid], jnp.maximum(cur, v), mask=is_last)
            return carry2

        lax.fori_loop(0, GPC, group_body, 0)

        for s in (0, 1):
            @pl.when(slot == s)
            def _():
                out_dma(chunk, s).start()

            @pl.when((slot == s) & (chunk + 2 < NCHK))
            def _():
                z_dma(chunk + 2, s).start()
                seg_dma(chunk + 2, s).start()
        return carry

    lax.fori_loop(0, NCHK, chunk_body, 0)

    out_dma(NCHK - 2, (NCHK - 2) % 2).wait()
    out_dma(NCHK - 1, (NCHK - 1) % 2).wait()
    pltpu.sync_copy(m_v, out_m_hbm.at[wid])


@functools.partial(
    pl.kernel,
    mesh=_MESH,
    compiler_params=_SC_PARAMS,
    out_type=jax.ShapeDtypeStruct((NBP,), jnp.float32),
    scratch_types=[
        pltpu.VMEM((NW, BPW), jnp.float32),
        pltpu.VMEM((BPW,), jnp.float32),
    ],
)
def _segmax_merge(parts_hbm, out_hbm, blk_v, acc_v):
    wid = lax.axis_index("s") * NC + lax.axis_index("c")
    lo = pl.multiple_of(wid * BPW, 8)
    pltpu.sync_copy(parts_hbm.at[:, pl.ds(lo, BPW)], blk_v)

    def body(j, carry):
        off = pl.multiple_of(j * L, L)
        acc = jnp.full((L,), NEG, jnp.float32)
        for r in range(NW):
            acc = jnp.maximum(acc, blk_v[r, pl.ds(off, L)])
        acc_v[pl.ds(off, L)] = acc
        return carry

    lax.fori_loop(0, BPW // L, body, 0)
    pltpu.sync_copy(acc_v, out_hbm.at[pl.ds(lo, BPW)])


def kernel(z_ins, bag_idx, W, b):
    seg = bag_idx.astype(jnp.int32)
    w1 = W.reshape(D)
    b16 = jnp.broadcast_to(b, (L,))
    scores, parts = _sc_fused(z_ins, seg, w1, b16)
    merged = _segmax_merge(parts)
    M = merged[:NB][:, None]
    return (M, None, scores)


# speedup vs baseline: 2.1906x; 1.0527x over previous
import functools

import jax
import jax.numpy as jnp
from jax import lax
from jax.experimental import pallas as pl
from jax.experimental.pallas import tpu as pltpu
from jax.experimental.pallas import tpu_sc as plsc

N = 320000
D = 128
NB = 10000

NC = 2
NS = 16
NW = NC * NS
C = N // NW
NBP = 10240
BPW = NBP // NW
L = 16

RCH = 400
NCHK = C // RCH
GPC = RCH // L

NEG = float("-inf")

_MESH = plsc.VectorSubcoreMesh(core_axis_name="c", subcore_axis_name="s")
_SC_PARAMS = pltpu.CompilerParams(
    needs_layout_passes=False, use_tc_tiling_on_sc=False)


def _take(v, idx):
    return jnp.take_along_axis(v, idx, axis=0, mode="promise_in_bounds")


@functools.partial(
    pl.kernel,
    mesh=_MESH,
    compiler_params=_SC_PARAMS,
    out_type=(
        jax.ShapeDtypeStruct((N,), jnp.float32),
        jax.ShapeDtypeStruct((NW, NBP), jnp.float32),
    ),
    scratch_types=[
        pltpu.VMEM((2 * RCH, D), jnp.float32),
        pltpu.VMEM((2, RCH), jnp.int32),
        pltpu.VMEM((2, RCH), jnp.float32),
        pltpu.VMEM((NBP,), jnp.float32),
        pltpu.VMEM((D,), jnp.float32),
        pltpu.VMEM((L,), jnp.float32),
        pltpu.SemaphoreType.DMA((2,)),
        pltpu.SemaphoreType.DMA((2,)),
        pltpu.SemaphoreType.DMA((2,)),
    ],
)
def _sc_fused(z_hbm, seg_hbm, w_hbm, b_hbm, out_s_hbm, out_m_hbm,
              zb, segb, sb, m_v, w_v, b_v, zsem, gsem, osem):
    wid = lax.axis_index("s") * NC + lax.axis_index("c")
    base = pl.multiple_of(wid * C, 8)

    pltpu.sync_copy(w_hbm, w_v)
    pltpu.sync_copy(b_hbm, b_v)

    def z_dma(chunk, slot):
        return pltpu.make_async_copy(
            z_hbm.at[pl.ds(base + chunk * RCH, RCH), :],
            zb.at[pl.ds(slot * RCH, RCH), :], zsem.at[slot])

    def seg_dma(chunk, slot):
        return pltpu.make_async_copy(
            seg_hbm.at[pl.ds(base + chunk * RCH, RCH)],
            segb.at[slot], gsem.at[slot])

    def out_dma(chunk, slot):
        return pltpu.make_async_copy(
            sb.at[slot],
            out_s_hbm.at[pl.ds(base + chunk * RCH, RCH)], osem.at[slot])

    z_dma(0, 0).start()
    seg_dma(0, 0).start()
    z_dma(1, 1).start()
    seg_dma(1, 1).start()

    neg = jnp.full((L,), NEG, jnp.float32)

    def init_body(i, carry):
        m_v[pl.ds(pl.multiple_of(i * L, L), L)] = neg
        return carry

    lax.fori_loop(0, NBP // L, init_body, 0, unroll=8)

    wv = [w_v[pl.ds(16 * j, L)] for j in range(D // L)]
    bvec = b_v[...]
    lane = lax.iota(jnp.int32, L)
    last_lane = lane == (L - 1)
    up1 = jnp.minimum(lane + 1, L - 1)

    def chunk_body(chunk, carry):
        slot = lax.rem(chunk, 2)

        for s in (0, 1):
            @pl.when(slot == s)
            def _():
                z_dma(chunk, s).wait()
                seg_dma(chunk, s).wait()

            @pl.when((slot == s) & (chunk >= 2))
            def _():
                out_dma(chunk - 2, s).wait()

        def group_body(gin, carry2):
            rowbase = slot * RCH + gin * L
            v = neg
            for l in range(L):
                row = rowbase + l
                prods = [zb[row, pl.ds(16 * j, L)] * wv[j]
                         for j in range(D // L)]
                while len(prods) > 1:
                    prods = [a + b for a, b in zip(prods[::2], prods[1::2])]
                v = jnp.where(lane == l, jnp.sum(prods[0]), v)
            v = v + bvec
            sb[slot, pl.ds(pl.multiple_of(gin * L, L), L)] = v

            gid = segb[slot, pl.ds(pl.multiple_of(gin * L, L), L)]
            for s in (1, 2, 4, 8):
                idx = jnp.maximum(lane - s, 0)
                vs = _take(v, idx)
                gs = _take(gid, idx)
                v = jnp.where((gs == gid) & (lane >= s),
                              jnp.maximum(v, vs), v)
            g_next = _take(gid, up1)
            is_last = (g_next != gid) | last_lane
            cur = plsc.load_gather(m_v, [gid], mask=is_last)
            plsc.store_scatter(m_v, [gid], jnp.maximum(cur, v), mask=is_last)
            return carry2

        lax.fori_loop(0, GPC, group_body, 0, unroll=2)

        for s in (0, 1):
            @pl.when(slot == s)
            def _():
                out_dma(chunk, s).start()

            @pl.when((slot == s) & (chunk + 2 < NCHK))
            def _():
                z_dma(chunk + 2, s).start()
                seg_dma(chunk + 2, s).start()
        return carry

    lax.fori_loop(0, NCHK, chunk_body, 0)

    out_dma(NCHK - 2, (NCHK - 2) % 2).wait()
    out_dma(NCHK - 1, (NCHK - 1) % 2).wait()
    pltpu.sync_copy(m_v, out_m_hbm.at[wid])


@functools.partial(
    pl.kernel,
    mesh=_MESH,
    compiler_params=_SC_PARAMS,
    out_type=jax.ShapeDtypeStruct((NBP,), jnp.float32),
    scratch_types=[
        pltpu.VMEM((NW, BPW), jnp.float32),
        pltpu.VMEM((BPW,), jnp.float32),
    ],
)
def _segmax_merge(parts_hbm, out_hbm, blk_v, acc_v):
    wid = lax.axis_index("s") * NC + lax.axis_index("c")
    lo = pl.multiple_of(wid * BPW, 8)
    pltpu.sync_copy(parts_hbm.at[:, pl.ds(lo, BPW)], blk_v)

    def body(j, carry):
        off = pl.multiple_of(j * L, L)
        acc = jnp.full((L,), NEG, jnp.float32)
        for r in range(NW):
            acc = jnp.maximum(acc, blk_v[r, pl.ds(off, L)])
        acc_v[pl.ds(off, L)] = acc
        return carry

    lax.fori_loop(0, BPW // L, body, 0)
    pltpu.sync_copy(acc_v, out_hbm.at[pl.ds(lo, BPW)])


def kernel(z_ins, bag_idx, W, b):
    seg = bag_idx.astype(jnp.int32)
    w1 = W.reshape(D)
    b16 = jnp.broadcast_to(b, (L,))
    scores, parts = _sc_fused(z_ins, seg, w1, b16)
    merged = _segmax_merge(parts)
    M = merged[:NB][:, None]
    return (M, None, scores)
